# baseline (speedup 1.0000x reference)
import functools

import jax
import jax.numpy as jnp
from jax import lax
from jax.experimental import pallas as pl
from jax.experimental.pallas import tpu as pltpu
from jax.experimental.pallas import tpu_sc as plsc

B, T, D = 16, 512, 2048
S = 1024
A_DEN, A_NUM = 16384, 4096
LEAKY = 1e-05
L = 16
LN2 = 0.6931471805599453


def _tree_sum(v, tmp_v, lane):
    for k in (1, 2, 4, 8):
        tmp_v[...] = v
        v = v + plsc.load_gather(tmp_v, [lane ^ k])
    return v


def _run_graph(b, mylen, x, src_h, dst_h, pdf_h, lp_h, init_h, final_h, out,
               ia_v, packed_v, w_v, p_v, np_v, pinit_v, pfinal_v, xrow_v,
               ex_v, tmp_v, xsem, *, A, leaky, graph):
    nch = A // L
    ns = S // L
    zero16f = jnp.zeros((L,), jnp.float32)
    one16f = jnp.ones((L,), jnp.float32)
    zero16i = jnp.zeros((L,), jnp.int32)
    lane = lax.iota(jnp.int32, L)

    pltpu.sync_copy(pdf_h, ia_v.at[pl.ds(0, A)])

    def pk_pdf(i, _):
        sl = pl.ds(i * L, L)
        packed_v[sl] = ia_v[sl] << 10
        return 0
    lax.fori_loop(0, nch, pk_pdf, 0)

    pltpu.sync_copy(src_h, ia_v.at[pl.ds(0, A)])

    def pk_src(i, _):
        sl = pl.ds(i * L, L)
        packed_v[sl] = packed_v[sl] | ia_v[sl]
        return 0
    lax.fori_loop(0, nch, pk_src, 0)

    pltpu.sync_copy(dst_h, ia_v.at[pl.ds(0, A)])

    def pk_dst(i, _):
        sl = pl.ds(i * L, L)
        packed_v[sl] = packed_v[sl] | (ia_v[sl] << 21)
        return 0
    lax.fori_loop(0, nch, pk_dst, 0)

    pltpu.sync_copy(lp_h, w_v.at[pl.ds(0, A)])

    def pk_w(i, _):
        sl = pl.ds(i * L, L)
        w_v[sl] = jnp.exp(w_v[sl])
        return 0
    lax.fori_loop(0, nch, pk_w, 0)

    pltpu.sync_copy(init_h, pinit_v)
    pltpu.sync_copy(final_h, pfinal_v)

    def pk_if(i, acc):
        sl = pl.ds(i * L, L)
        v = jnp.exp(pinit_v[sl])
        pinit_v[sl] = v
        p_v[sl] = v
        np_v[sl] = zero16f
        pfinal_v[sl] = jnp.exp(pfinal_v[sl])
        return acc + v
    acc0 = lax.fori_loop(0, ns, pk_if, zero16f)
    sum0 = _tree_sum(acc0, tmp_v, lane)

    pltpu.make_async_copy(x.at[b, 0], xrow_v.at[0], xsem).start()

    def step(t, carry):
        sumv, scv, Epend, Etot = carry
        Etot = Etot + Epend
        par = t & 1
        pltpu.make_async_copy(x.at[b, t], xrow_v.at[par], xsem).wait()
        nt = jnp.minimum(t + 1, T - 1)
        pltpu.make_async_copy(x.at[b, nt], xrow_v.at[1 - par], xsem).start()

        @plsc.parallel_loop(0, D // L, unroll=8)
        def exb(j):
            sl = pl.ds(j * L, L)
            ex_v[sl] = jnp.exp(xrow_v[par, sl]) * scv

        if leaky > 0.0:
            klk = sumv * scv * leaky
        else:
            klk = zero16f

        @plsc.parallel_loop(0, nch, unroll=16)
        def ab(i):
            sl = pl.ds(i * L, L)
            pk = packed_v[sl]
            wv = w_v[sl]
            isrc = pk & 1023
            ipdf = lax.shift_right_logical(pk, 10) & 2047
            idst = lax.shift_right_logical(pk, 21)
            pg = plsc.load_gather(p_v, [isrc])
            eg = plsc.load_gather(ex_v, [ipdf])
            plsc.addupdate_scatter(np_v, [idst], pg * wv * eg)

        @plsc.parallel_loop(0, ns, unroll=8, carry=zero16f)
        def acc(j, a):
            sl = pl.ds(j * L, L)
            u = np_v[sl] + klk * pinit_v[sl]
            p_v[sl] = u
            np_v[sl] = zero16f
            return a + u

        tot = _tree_sum(acc, tmp_v, lane)
        ok = tot > 0.0
        bits = plsc.bitcast(tot, jnp.int32)
        Ev = lax.shift_right_logical(bits, 23) - 127
        Ev = jnp.where(ok, Ev, zero16i)
        scnew = jnp.where(ok, plsc.bitcast((127 - Ev) << 23, jnp.float32),
                          one16f)
        return (tot, scnew, Ev, Etot)

    carry0 = (sum0, one16f, zero16i, zero16i)
    _, _, _, Etot = lax.fori_loop(0, mylen, step, carry0)

    pltpu.make_async_copy(x.at[b, 0], xrow_v.at[0], xsem).wait()

    def vb(j, acc):
        sl = pl.ds(j * L, L)
        return acc + p_v[sl] * pfinal_v[sl]
    acc = lax.fori_loop(0, ns, vb, zero16f)

    r = 2 * b + graph
    tmp_v[...] = acc
    pltpu.sync_copy(tmp_v, out.at[r, 0])
    tmp_v[...] = Etot.astype(jnp.float32)
    pltpu.sync_copy(tmp_v, out.at[r, 1])


def _sc_body(x, lengths, dsrc, ddst, dpdf, dlp, dinit, dfinal,
             nsrc, ndst, npdf, nlp, ninit, nfinal, out,
             ia_v, packed_v, w_v, p_v, np_v, pinit_v, pfinal_v,
             xrow_v, ex_v, len_v, tmp_v, xsem):
    core = lax.axis_index("c")
    b = lax.axis_index("s")

    pltpu.sync_copy(lengths, len_v)
    lenv = len_v[...]
    lane = lax.iota(jnp.int32, L)
    mine = lane == b
    mylen = jnp.int32(0)
    for k in range(10):
        bit = jnp.any(mine & (((lenv >> k) & 1) == 1))
        mylen = mylen + jnp.where(bit, jnp.int32(1 << k), jnp.int32(0))

    common = (ia_v, packed_v, w_v, p_v, np_v, pinit_v, pfinal_v, xrow_v,
              ex_v, tmp_v, xsem)

    @pl.when(core == 0)
    def _():
        _run_graph(b, mylen, x, dsrc, ddst, dpdf, dlp, dinit, dfinal, out,
                   *common, A=A_DEN, leaky=LEAKY, graph=0)

    @pl.when(core == 1)
    def _():
        _run_graph(b, mylen, x, nsrc, ndst, npdf, nlp, ninit, nfinal, out,
                   *common, A=A_NUM, leaky=0.0, graph=1)


_sc_call = pl.kernel(
    _sc_body,
    out_type=jax.ShapeDtypeStruct((2 * B, 2, L), jnp.float32),
    mesh=plsc.VectorSubcoreMesh(core_axis_name="c", subcore_axis_name="s"),
    compiler_params=pltpu.CompilerParams(needs_layout_passes=False),
    scratch_types=[
        pltpu.VMEM((A_DEN,), jnp.int32),
        pltpu.VMEM((A_DEN,), jnp.int32),
        pltpu.VMEM((A_DEN,), jnp.float32),
        pltpu.VMEM((S,), jnp.float32),
        pltpu.VMEM((S,), jnp.float32),
        pltpu.VMEM((S,), jnp.float32),
        pltpu.VMEM((S,), jnp.float32),
        pltpu.VMEM((2, D), jnp.float32),
        pltpu.VMEM((D,), jnp.float32),
        pltpu.VMEM((L,), jnp.int32),
        pltpu.VMEM((L,), jnp.float32),
        pltpu.SemaphoreType.DMA,
    ],
)


def _epi_body(om_ref, len_ref, o_ref):
    om = om_ref[...]
    val = jnp.sum(om[:, 0, :], axis=-1)
    lg = jnp.log(val) + om[:, 1, 0] * LN2
    row = lax.iota(jnp.int32, 2 * B)
    coef = jnp.where(row % 2 == 0, 1.0, -1.0)
    tot = jnp.sum(lg * coef)
    sl = jnp.sum(len_ref[...].astype(jnp.float32))
    o_ref[...] = jnp.broadcast_to(tot / sl, (1, 1))


_epi_call = pl.pallas_call(
    _epi_body,
    out_shape=jax.ShapeDtypeStruct((1, 1), jnp.float32),
)


@jax.jit
def kernel(x, x_lengths, den_src, den_dst, den_pdf, den_arc_lp, den_init,
           den_final, num_src, num_dst, num_pdf, num_arc_lp, num_init,
           num_final):
    res = _sc_call(x, x_lengths, den_src, den_dst, den_pdf, den_arc_lp,
                   den_init, den_final, num_src, num_dst, num_pdf,
                   num_arc_lp, num_init, num_final)
    out = _epi_call(res, x_lengths.reshape(1, B))
    return out[0, 0]


# speedup vs baseline: 1.1589x; 1.1589x over previous
import functools

import jax
import jax.numpy as jnp
from jax import lax
from jax.experimental import pallas as pl
from jax.experimental.pallas import tpu as pltpu
from jax.experimental.pallas import tpu_sc as plsc

B, T, D = 16, 512, 2048
S = 1024
A_DEN, A_NUM = 16384, 4096
LEAKY = 1e-05
L = 16
LN2 = 0.6931471805599453


def _tree_sum(v, tmp_v, lane):
    for k in (1, 2, 4, 8):
        tmp_v[...] = v
        v = v + plsc.load_gather(tmp_v, [lane ^ k])
    return v


def _run_graph(b, mylen, x, src_h, dst_h, pdf_h, lp_h, init_h, final_h, out,
               ia_v, packed_v, w_v, p_v, np_v, pinit_v, pfinal_v, xrow_v,
               ex_v, tmp_v, xsem, *, A, leaky, graph):
    nch = A // L
    ns = S // L
    zero16f = jnp.zeros((L,), jnp.float32)
    one16f = jnp.ones((L,), jnp.float32)
    zero16i = jnp.zeros((L,), jnp.int32)
    lane = lax.iota(jnp.int32, L)

    pltpu.sync_copy(pdf_h, ia_v.at[pl.ds(0, A)])

    def pk_pdf(i, _):
        sl = pl.ds(i * L, L)
        packed_v[sl] = ia_v[sl] << 10
        return 0
    lax.fori_loop(0, nch, pk_pdf, 0)

    pltpu.sync_copy(src_h, ia_v.at[pl.ds(0, A)])

    def pk_src(i, _):
        sl = pl.ds(i * L, L)
        packed_v[sl] = packed_v[sl] | ia_v[sl]
        return 0
    lax.fori_loop(0, nch, pk_src, 0)

    pltpu.sync_copy(dst_h, ia_v.at[pl.ds(0, A)])

    def pk_dst(i, _):
        sl = pl.ds(i * L, L)
        packed_v[sl] = packed_v[sl] | (ia_v[sl] << 21)
        return 0
    lax.fori_loop(0, nch, pk_dst, 0)

    pltpu.sync_copy(lp_h, w_v.at[pl.ds(0, A)])

    def pk_w(i, _):
        sl = pl.ds(i * L, L)
        w_v[sl] = jnp.exp(w_v[sl])
        return 0
    lax.fori_loop(0, nch, pk_w, 0)

    pltpu.sync_copy(init_h, pinit_v)
    pltpu.sync_copy(final_h, pfinal_v)

    def pk_if(i, acc):
        sl = pl.ds(i * L, L)
        v = jnp.exp(pinit_v[sl])
        pinit_v[sl] = v
        p_v[sl] = v
        np_v[sl] = zero16f
        pfinal_v[sl] = jnp.exp(pfinal_v[sl])
        return acc + v
    acc0 = lax.fori_loop(0, ns, pk_if, zero16f)
    sum0 = _tree_sum(acc0, tmp_v, lane)

    pltpu.make_async_copy(x.at[b, 0], xrow_v.at[0], xsem).start()

    def step(t, carry):
        sumv, scv, Epend, Etot = carry
        Etot = Etot + Epend
        par = t & 1
        pltpu.make_async_copy(x.at[b, t], xrow_v.at[par], xsem).wait()
        nt = jnp.minimum(t + 1, T - 1)
        pltpu.make_async_copy(x.at[b, nt], xrow_v.at[1 - par], xsem).start()

        @plsc.parallel_loop(0, D // L, unroll=8)
        def exb(j):
            sl = pl.ds(j * L, L)
            ex_v[sl] = jnp.exp(xrow_v[par, sl]) * scv

        if leaky > 0.0:
            klk = sumv * scv * leaky
        else:
            klk = zero16f

        @plsc.parallel_loop(0, nch, unroll=8)
        def ab(i):
            sl = pl.ds(i * L, L)
            pk = packed_v[sl]
            wv = w_v[sl]
            isrc = pk & 1023
            ipdf = lax.shift_right_logical(pk, 10) & 2047
            idst = lax.shift_right_logical(pk, 21)
            pg = plsc.load_gather(p_v, [isrc])
            eg = plsc.load_gather(ex_v, [ipdf])
            plsc.addupdate_scatter(np_v, [idst], pg * wv * eg)

        @plsc.parallel_loop(0, ns, unroll=8, carry=zero16f)
        def acc(j, a):
            sl = pl.ds(j * L, L)
            u = np_v[sl] + klk * pinit_v[sl]
            p_v[sl] = u
            np_v[sl] = zero16f
            return a + u

        tot = _tree_sum(acc, tmp_v, lane)
        ok = tot > 0.0
        bits = plsc.bitcast(tot, jnp.int32)
        Ev = lax.shift_right_logical(bits, 23) - 127
        Ev = jnp.where(ok, Ev, zero16i)
        scnew = jnp.where(ok, plsc.bitcast((127 - Ev) << 23, jnp.float32),
                          one16f)
        return (tot, scnew, Ev, Etot)

    carry0 = (sum0, one16f, zero16i, zero16i)
    _, _, _, Etot = lax.fori_loop(0, mylen, step, carry0)

    pltpu.make_async_copy(x.at[b, 0], xrow_v.at[0], xsem).wait()

    def vb(j, acc):
        sl = pl.ds(j * L, L)
        return acc + p_v[sl] * pfinal_v[sl]
    acc = lax.fori_loop(0, ns, vb, zero16f)

    r = 2 * b + graph
    tmp_v[...] = acc
    pltpu.sync_copy(tmp_v, out.at[r, 0])
    tmp_v[...] = Etot.astype(jnp.float32)
    pltpu.sync_copy(tmp_v, out.at[r, 1])


def _sc_body(x, lengths, dsrc, ddst, dpdf, dlp, dinit, dfinal,
             nsrc, ndst, npdf, nlp, ninit, nfinal, out,
             ia_v, packed_v, w_v, p_v, np_v, pinit_v, pfinal_v,
             xrow_v, ex_v, len_v, tmp_v, xsem):
    core = lax.axis_index("c")
    b = lax.axis_index("s")

    pltpu.sync_copy(lengths, len_v)
    lenv = len_v[...]
    lane = lax.iota(jnp.int32, L)
    mine = lane == b
    mylen = jnp.int32(0)
    for k in range(10):
        bit = jnp.any(mine & (((lenv >> k) & 1) == 1))
        mylen = mylen + jnp.where(bit, jnp.int32(1 << k), jnp.int32(0))

    common = (ia_v, packed_v, w_v, p_v, np_v, pinit_v, pfinal_v, xrow_v,
              ex_v, tmp_v, xsem)

    @pl.when(core == 0)
    def _():
        _run_graph(b, mylen, x, dsrc, ddst, dpdf, dlp, dinit, dfinal, out,
                   *common, A=A_DEN, leaky=LEAKY, graph=0)

    @pl.when(core == 1)
    def _():
        _run_graph(b, mylen, x, nsrc, ndst, npdf, nlp, ninit, nfinal, out,
                   *common, A=A_NUM, leaky=0.0, graph=1)


_sc_call = pl.kernel(
    _sc_body,
    out_type=jax.ShapeDtypeStruct((2 * B, 2, L), jnp.float32),
    mesh=plsc.VectorSubcoreMesh(core_axis_name="c", subcore_axis_name="s"),
    compiler_params=pltpu.CompilerParams(needs_layout_passes=False),
    scratch_types=[
        pltpu.VMEM((A_DEN,), jnp.int32),
        pltpu.VMEM((A_DEN,), jnp.int32),
        pltpu.VMEM((A_DEN,), jnp.float32),
        pltpu.VMEM((S,), jnp.float32),
        pltpu.VMEM((S,), jnp.float32),
        pltpu.VMEM((S,), jnp.float32),
        pltpu.VMEM((S,), jnp.float32),
        pltpu.VMEM((2, D), jnp.float32),
        pltpu.VMEM((D,), jnp.float32),
        pltpu.VMEM((L,), jnp.int32),
        pltpu.VMEM((L,), jnp.float32),
        pltpu.SemaphoreType.DMA,
    ],
)


def _epi_body(om_ref, len_ref, o_ref):
    om = om_ref[...]
    val = jnp.sum(om[:, 0, :], axis=-1)
    lg = jnp.log(val) + om[:, 1, 0] * LN2
    row = lax.iota(jnp.int32, 2 * B)
    coef = jnp.where(row % 2 == 0, 1.0, -1.0)
    tot = jnp.sum(lg * coef)
    sl = jnp.sum(len_ref[...].astype(jnp.float32))
    o_ref[...] = jnp.broadcast_to(tot / sl, (1, 1))


_epi_call = pl.pallas_call(
    _epi_body,
    out_shape=jax.ShapeDtypeStruct((1, 1), jnp.float32),
)


def _spread(src, dst, pdf, lp):
    a = src.shape[0]
    perm = jnp.argsort(dst % L, stable=True)
    perm = perm.reshape(L, a // L).T.reshape(-1)
    return src[perm], dst[perm], pdf[perm], lp[perm]


@jax.jit
def kernel(x, x_lengths, den_src, den_dst, den_pdf, den_arc_lp, den_init,
           den_final, num_src, num_dst, num_pdf, num_arc_lp, num_init,
           num_final):
    den_src, den_dst, den_pdf, den_arc_lp = _spread(
        den_src, den_dst, den_pdf, den_arc_lp)
    num_src, num_dst, num_pdf, num_arc_lp = _spread(
        num_src, num_dst, num_pdf, num_arc_lp)
    res = _sc_call(x, x_lengths, den_src, den_dst, den_pdf, den_arc_lp,
                   den_init, den_final, num_src, num_dst, num_pdf,
                   num_arc_lp, num_init, num_final)
    out = _epi_call(res, x_lengths.reshape(1, B))
    return out[0, 0]
